# Initial kernel scaffold; baseline (speedup 1.0000x reference)
import jax
import jax.numpy as jnp
from jax.experimental import pallas as pl


def kernel(input_ids, embed_table, W, b):
    raise NotImplementedError("write your pallas kernel here")


# speedup vs baseline: 1.6707x; 1.6707x over previous
import functools

import jax
import jax.numpy as jnp
from jax import lax
from jax.experimental import pallas as pl
from jax.experimental.pallas import tpu as pltpu
from jax.experimental.pallas import tpu_sc as plsc

VOCAB = 100000
EMBED = 32
B = 1024
L = 200

NC = 2
NS = 16
NW = NC * NS
BPW = B // NW
CHUNK = 100
CPR = L // CHUNK
NCHUNK = BPW * CPR


def _make_pool():
    mesh = plsc.VectorSubcoreMesh(core_axis_name="c", subcore_axis_name="s")

    @functools.partial(
        pl.kernel,
        mesh=mesh,
        compiler_params=pltpu.CompilerParams(use_tc_tiling_on_sc=False),
        out_type=jax.ShapeDtypeStruct((B, EMBED), jnp.float32),
        scratch_types=[
            pltpu.VMEM((NCHUNK, CHUNK), jnp.int32),
            pltpu.VMEM((CHUNK, EMBED), jnp.float32),
            pltpu.VMEM((BPW, EMBED), jnp.float32),
            pltpu.SemaphoreType.DMA,
        ],
    )
    def pool(ids_hbm, table_hbm, out_hbm, idx_v, rows_v, out_v, sem):
        wid = lax.axis_index("s") * NC + lax.axis_index("c")
        pltpu.sync_copy(ids_hbm.at[wid], idx_v)

        def row_body(i, carry):
            def chunk_body(k, accs):
                a0, a1 = accs
                pltpu.async_copy(
                    table_hbm.at[idx_v.at[i * CPR + k]], rows_v, sem
                ).wait()

                def tok_body(t, accs2):
                    b0, b1 = accs2
                    return (b0 + rows_v[t, pl.ds(0, 16)],
                            b1 + rows_v[t, pl.ds(16, 16)])

                return lax.fori_loop(0, CHUNK, tok_body, (a0, a1), unroll=10)

            z = jnp.zeros((16,), jnp.float32)
            a0, a1 = lax.fori_loop(0, CPR, chunk_body, (z, z))
            out_v[i, pl.ds(0, 16)] = a0
            out_v[i, pl.ds(16, 16)] = a1
            return carry

        lax.fori_loop(0, BPW, row_body, 0)
        pltpu.sync_copy(out_v, out_hbm.at[pl.ds(wid * BPW, BPW)])

    return pool


_pool = _make_pool()

BN = 2048
GRID_N = (VOCAB + BN - 1) // BN


def _matmul_body(x_ref, w_ref, b_ref, o_ref):
    x = x_ref[...] * (1.0 / L)
    o_ref[...] = (
        jnp.dot(x, w_ref[...], preferred_element_type=jnp.float32) + b_ref[...]
    )


def _matmul(pooled, w, b2):
    return pl.pallas_call(
        _matmul_body,
        grid=(GRID_N,),
        in_specs=[
            pl.BlockSpec((B, EMBED), lambda n: (0, 0)),
            pl.BlockSpec((EMBED, BN), lambda n: (0, n)),
            pl.BlockSpec((1, BN), lambda n: (0, n)),
        ],
        out_specs=pl.BlockSpec((B, BN), lambda n: (0, n)),
        out_shape=jax.ShapeDtypeStruct((B, VOCAB), jnp.float32),
    )(pooled, w, b2)


def kernel(input_ids, embed_table, W, b):
    ids3 = input_ids.reshape(NW, NCHUNK, CHUNK)
    pooled = _pool(ids3, embed_table)
    logits = _matmul(pooled, W, b.reshape(1, VOCAB))
    return logits[:, None, :]
